# Initial kernel scaffold; baseline (speedup 1.0000x reference)
import functools

import jax
import jax.numpy as jnp
from jax import lax
from jax.experimental import pallas as pl
from jax.experimental.pallas import tpu as pltpu
from jax.experimental.pallas import tpu_sc as plsc

B = 16384
L = 20
EMB = 32
NC = 2
NS = 16
NW = NC * NS
RPW = B // NW
CH = 128
NCH = RPW // CH
GW = 128
GPC = CH * L // GW
IDX_ROWS = B * L // GW
IDX_RPW = IDX_ROWS // NW


def _body(anime_hbm, tokidx_hbm, tokpad_hbm, name_hbm, text_hbm, out_hbm,
          aidx_v, tidx_v, tpad_v, trows_v, nrows_v, outb_v, t0_v, gsem, osem):
    wid = lax.axis_index("s") * NC + lax.axis_index("c")
    base = wid * RPW

    pltpu.sync_copy(anime_hbm.at[pl.ds(base, RPW)], aidx_v)
    pltpu.sync_copy(tokidx_hbm.at[pl.ds(wid * IDX_RPW, IDX_RPW)], tidx_v)
    pltpu.sync_copy(tokpad_hbm.at[pl.ds(base, RPW)], tpad_v)
    pltpu.sync_copy(text_hbm.at[pl.ds(0, 1)], t0_v)

    t0a = t0_v[0, pl.ds(0, 16)]
    t0b = t0_v[0, pl.ds(16, 16)]
    lf = jnp.full((16,), float(L), dtype=jnp.float32)
    one = jnp.full((16,), 1, dtype=jnp.int32)

    for c in range(NCH):
        dmas = [pltpu.async_copy(
            name_hbm.at[aidx_v.at[pl.ds(c * CH, CH)]], nrows_v, gsem)]
        for j in range(GPC):
            dmas.append(pltpu.async_copy(
                text_hbm.at[tidx_v.at[c * GPC + j]],
                trows_v.at[pl.ds(j * GW, GW)], gsem))
        for d in dmas:
            d.wait()

        def row(r, _):
            gr = c * CH + r
            ids_a = tpad_v[gr, pl.ds(0, 16)]
            ids_b = tpad_v[gr, pl.ds(16, 16)]
            nnz = (plsc.all_reduce_population_count(ids_a != 0)
                   + plsc.all_reduce_population_count(ids_b != 0))
            tb = r * L
            acc_a = trows_v[tb, pl.ds(0, 16)]
            acc_b = trows_v[tb, pl.ds(16, 16)]
            for t in range(1, L):
                acc_a = acc_a + trows_v[tb + t, pl.ds(0, 16)]
                acc_b = acc_b + trows_v[tb + t, pl.ds(16, 16)]
            nnzf = nnz.astype(jnp.float32)
            n0f = lf - nnzf
            recip = 1.0 / jnp.maximum(nnz, one).astype(jnp.float32)
            outb_v[r, pl.ds(0, 16)] = nrows_v[r, pl.ds(0, 16)]
            outb_v[r, pl.ds(16, 16)] = nrows_v[r, pl.ds(16, 16)]
            outb_v[r, pl.ds(32, 16)] = (acc_a - n0f * t0a) * recip
            outb_v[r, pl.ds(48, 16)] = (acc_b - n0f * t0b) * recip
            return 0

        lax.fori_loop(0, CH, row, 0)
        pltpu.sync_copy(outb_v, out_hbm.at[pl.ds(base + c * CH, CH)])


def kernel(anime_ids, token_ids, name_table, text_table):
    anime_ids = anime_ids.astype(jnp.int32)
    tok32 = token_ids.astype(jnp.int32)
    tok_idx = tok32.reshape(IDX_ROWS, GW)
    tok_pad = jnp.pad(tok32, ((0, 0), (0, 32 - L)))

    mesh = plsc.VectorSubcoreMesh(core_axis_name="c", subcore_axis_name="s")
    run = functools.partial(
        pl.kernel, mesh=mesh,
        out_type=jax.ShapeDtypeStruct((B, 2 * EMB), jnp.float32),
        scratch_types=[
            pltpu.VMEM((RPW,), jnp.int32),
            pltpu.VMEM((IDX_RPW, GW), jnp.int32),
            pltpu.VMEM((RPW, 32), jnp.int32),
            pltpu.VMEM((CH * L, EMB), jnp.float32),
            pltpu.VMEM((CH, EMB), jnp.float32),
            pltpu.VMEM((CH, 2 * EMB), jnp.float32),
            pltpu.VMEM((1, EMB), jnp.float32),
            pltpu.SemaphoreType.DMA,
            pltpu.SemaphoreType.DMA,
        ],
    )(_body)
    return run(anime_ids, tok_idx, tok_pad, name_table, text_table)


# speedup vs baseline: 12.2769x; 12.2769x over previous
import functools

import jax
import jax.numpy as jnp
from jax import lax
from jax.experimental import pallas as pl
from jax.experimental.pallas import tpu as pltpu
from jax.experimental.pallas import tpu_sc as plsc

B = 16384
L = 20
EMB = 32
NC = 2
NS = 16
NW = NC * NS
RPW = B // NW
CH = 128
NCH = RPW // CH
GW = 128
GPC = CH * L // GW
IDX_ROWS = B * L // GW
IDX_RPW = IDX_ROWS // NW


def _body(anime_hbm, tokidx_hbm, tokpad_hbm, name_hbm, text_hbm, out_hbm,
          aidx_v, tidx_v, tpad_v, trows_v, nrows_v, outb_v, t0_v, gsem, osem):
    wid = lax.axis_index("s") * NC + lax.axis_index("c")
    base = wid * RPW

    pltpu.sync_copy(anime_hbm.at[pl.ds(base, RPW)], aidx_v)
    pltpu.sync_copy(tokidx_hbm.at[pl.ds(wid * IDX_RPW, IDX_RPW)], tidx_v)
    pltpu.sync_copy(tokpad_hbm.at[pl.ds(base, RPW)], tpad_v)
    pltpu.sync_copy(text_hbm.at[pl.ds(0, 1)], t0_v)

    t0a = t0_v[0, pl.ds(0, 16)]
    t0b = t0_v[0, pl.ds(16, 16)]
    lf = jnp.full((16,), float(L), dtype=jnp.float32)
    one = jnp.full((16,), 1, dtype=jnp.int32)

    for c in range(NCH):
        dmas = [pltpu.async_copy(
            name_hbm.at[aidx_v.at[pl.ds(c * CH, CH)]], nrows_v, gsem)]
        for j in range(GPC):
            dmas.append(pltpu.async_copy(
                text_hbm.at[tidx_v.at[c * GPC + j]],
                trows_v.at[pl.ds(j * GW, GW)], gsem))
        for d in dmas:
            d.wait()

        def row(r, _):
            gr = c * CH + r
            ids_a = tpad_v[gr, pl.ds(0, 16)]
            ids_b = tpad_v[gr, pl.ds(16, 16)]
            nnz = (jnp.sum(jnp.where(ids_a != 0, 1, 0))
                   + jnp.sum(jnp.where(ids_b != 0, 1, 0)))
            tb = r * L
            acc_a = trows_v[tb, pl.ds(0, 16)]
            acc_b = trows_v[tb, pl.ds(16, 16)]
            for t in range(1, L):
                acc_a = acc_a + trows_v[tb + t, pl.ds(0, 16)]
                acc_b = acc_b + trows_v[tb + t, pl.ds(16, 16)]
            nnzf = nnz.astype(jnp.float32)
            n0f = lf - nnzf
            recip = 1.0 / jnp.maximum(nnz, one).astype(jnp.float32)
            outb_v[r, pl.ds(0, 16)] = nrows_v[r, pl.ds(0, 16)]
            outb_v[r, pl.ds(16, 16)] = nrows_v[r, pl.ds(16, 16)]
            outb_v[r, pl.ds(32, 16)] = (acc_a - n0f * t0a) * recip
            outb_v[r, pl.ds(48, 16)] = (acc_b - n0f * t0b) * recip
            return 0

        lax.fori_loop(0, CH, row, 0)
        pltpu.sync_copy(outb_v, out_hbm.at[pl.ds(base + c * CH, CH)])


def kernel(anime_ids, token_ids, name_table, text_table):
    anime_ids = anime_ids.astype(jnp.int32)
    tok32 = token_ids.astype(jnp.int32)
    tok_idx = tok32.reshape(IDX_ROWS, GW)
    tok_pad = jnp.pad(tok32, ((0, 0), (0, 32 - L)))

    mesh = plsc.VectorSubcoreMesh(core_axis_name="c", subcore_axis_name="s")
    run = functools.partial(
        pl.kernel, mesh=mesh,
        out_type=jax.ShapeDtypeStruct((B, 2 * EMB), jnp.float32),
        compiler_params=pltpu.CompilerParams(
            needs_layout_passes=False, use_tc_tiling_on_sc=False),
        scratch_types=[
            pltpu.VMEM((RPW,), jnp.int32),
            pltpu.VMEM((IDX_RPW, GW), jnp.int32),
            pltpu.VMEM((RPW, 32), jnp.int32),
            pltpu.VMEM((CH * L, EMB), jnp.float32),
            pltpu.VMEM((CH, EMB), jnp.float32),
            pltpu.VMEM((CH, 2 * EMB), jnp.float32),
            pltpu.VMEM((1, EMB), jnp.float32),
            pltpu.SemaphoreType.DMA,
            pltpu.SemaphoreType.DMA,
        ],
    )(_body)
    return run(anime_ids, tok_idx, tok_pad, name_table, text_table)


# speedup vs baseline: 13.1641x; 1.0723x over previous
import functools

import jax
import jax.numpy as jnp
from jax import lax
from jax.experimental import pallas as pl
from jax.experimental.pallas import tpu as pltpu
from jax.experimental.pallas import tpu_sc as plsc

B = 16384
L = 20
EMB = 32
NC = 2
NS = 16
NW = NC * NS
RPW = B // NW
CH = 64
NCH = RPW // CH
GW = 128
GPC = CH * L // GW
IDX_ROWS = B * L // GW
IDX_RPW = IDX_ROWS // NW


def _tree_sum(vs):
    while len(vs) > 1:
        nxt = [vs[i] + vs[i + 1] for i in range(0, len(vs) - 1, 2)]
        if len(vs) % 2:
            nxt.append(vs[-1])
        vs = nxt
    return vs[0]


def _body(anime_hbm, tokidx_hbm, tokpad_hbm, name_hbm, text_hbm, out_hbm,
          aidx_v, tidx_v, tpad_v, trows0, trows1, nrows0, nrows1,
          outb0, outb1, t0_v, gsem0, gsem1, osem0, osem1):
    wid = lax.axis_index("s") * NC + lax.axis_index("c")
    base = wid * RPW

    pltpu.sync_copy(anime_hbm.at[pl.ds(base, RPW)], aidx_v)
    pltpu.sync_copy(tokidx_hbm.at[pl.ds(wid * IDX_RPW, IDX_RPW)], tidx_v)
    pltpu.sync_copy(tokpad_hbm.at[pl.ds(base, RPW)], tpad_v)
    pltpu.sync_copy(text_hbm.at[pl.ds(0, 1)], t0_v)

    t0a = t0_v[0, pl.ds(0, 16)]
    t0b = t0_v[0, pl.ds(16, 16)]
    lf = jnp.full((16,), float(L), dtype=jnp.float32)
    one = jnp.full((16,), 1, dtype=jnp.int32)

    bufs = [(trows0, nrows0, outb0, gsem0, osem0),
            (trows1, nrows1, outb1, gsem1, osem1)]

    def fire(c):
        trows, nrows, _, gsem, _ = bufs[c % 2]
        ds = [pltpu.async_copy(
            name_hbm.at[aidx_v.at[pl.ds(c * CH, CH)]], nrows, gsem)]
        for j in range(GPC):
            ds.append(pltpu.async_copy(
                text_hbm.at[tidx_v.at[c * GPC + j]],
                trows.at[pl.ds(j * GW, GW)], gsem))
        return ds

    pending = fire(0)
    out_dmas = [None, None]
    for c in range(NCH):
        nxt = fire(c + 1) if c + 1 < NCH else []
        for d in pending:
            d.wait()
        pending = nxt
        trows, nrows, outb, _, osem = bufs[c % 2]
        if out_dmas[c % 2] is not None:
            out_dmas[c % 2].wait()

        @plsc.parallel_loop(0, CH, unroll=2)
        def row(r):
            gr = c * CH + r
            ids_a = tpad_v[gr, pl.ds(0, 16)]
            ids_b = tpad_v[gr, pl.ds(16, 16)]
            nnz = (jnp.sum(jnp.where(ids_a != 0, 1, 0))
                   + jnp.sum(jnp.where(ids_b != 0, 1, 0)))
            tb = r * L
            acc_a = _tree_sum([trows[tb + t, pl.ds(0, 16)] for t in range(L)])
            acc_b = _tree_sum([trows[tb + t, pl.ds(16, 16)] for t in range(L)])
            n0f = lf - nnz.astype(jnp.float32)
            recip = 1.0 / jnp.maximum(nnz, one).astype(jnp.float32)
            outb[r, pl.ds(0, 16)] = nrows[r, pl.ds(0, 16)]
            outb[r, pl.ds(16, 16)] = nrows[r, pl.ds(16, 16)]
            outb[r, pl.ds(32, 16)] = (acc_a - n0f * t0a) * recip
            outb[r, pl.ds(48, 16)] = (acc_b - n0f * t0b) * recip

        out_dmas[c % 2] = pltpu.async_copy(
            outb, out_hbm.at[pl.ds(base + c * CH, CH)], osem)
    for d in out_dmas:
        if d is not None:
            d.wait()


def kernel(anime_ids, token_ids, name_table, text_table):
    anime_ids = anime_ids.astype(jnp.int32)
    tok32 = token_ids.astype(jnp.int32)
    tok_idx = tok32.reshape(IDX_ROWS, GW)
    tok_pad = jnp.pad(tok32, ((0, 0), (0, 32 - L)))

    mesh = plsc.VectorSubcoreMesh(core_axis_name="c", subcore_axis_name="s")
    run = functools.partial(
        pl.kernel, mesh=mesh,
        out_type=jax.ShapeDtypeStruct((B, 2 * EMB), jnp.float32),
        compiler_params=pltpu.CompilerParams(
            needs_layout_passes=False, use_tc_tiling_on_sc=False),
        scratch_types=[
            pltpu.VMEM((RPW,), jnp.int32),
            pltpu.VMEM((IDX_RPW, GW), jnp.int32),
            pltpu.VMEM((RPW, 32), jnp.int32),
            pltpu.VMEM((CH * L, EMB), jnp.float32),
            pltpu.VMEM((CH * L, EMB), jnp.float32),
            pltpu.VMEM((CH, EMB), jnp.float32),
            pltpu.VMEM((CH, EMB), jnp.float32),
            pltpu.VMEM((CH, 2 * EMB), jnp.float32),
            pltpu.VMEM((CH, 2 * EMB), jnp.float32),
            pltpu.VMEM((1, EMB), jnp.float32),
            pltpu.SemaphoreType.DMA,
            pltpu.SemaphoreType.DMA,
            pltpu.SemaphoreType.DMA,
            pltpu.SemaphoreType.DMA,
        ],
    )(_body)
    return run(anime_ids, tok_idx, tok_pad, name_table, text_table)


# speedup vs baseline: 17.0780x; 1.2973x over previous
import functools

import jax
import jax.numpy as jnp
from jax import lax
from jax.experimental import pallas as pl
from jax.experimental.pallas import tpu as pltpu
from jax.experimental.pallas import tpu_sc as plsc

B = 16384
L = 20
EMB = 32
VOCAB1 = 100001
VPAD = 100008
TVOCAB = 10000
NC = 2
NS = 16
NW = NC * NS

P1_CH = 2048
CG = 4
CPG = EMB // CG
RG = NW // CG
RPG = B // RG
P2_CH = 512
N2CH = RPG // P2_CH


def _body(aidx_hbm, idsT_hbm, nameT_hbm, textT_hbm, outT_hbm,
          big_v, ids_v, aidx_v, oute1_v, acc_v, sem, osem):
    wid = lax.axis_index("s") * NC + lax.axis_index("c")

    pltpu.sync_copy(nameT_hbm.at[wid], big_v)

    for c0 in range(0, B, P1_CH):
        pltpu.sync_copy(aidx_hbm.at[pl.ds(c0, P1_CH)], aidx_v)

        @plsc.parallel_loop(0, P1_CH // 16, unroll=2)
        def e1grp(g):
            rr = g * 16
            idx = aidx_v[pl.ds(rr, 16)]
            oute1_v[pl.ds(rr, 16)] = plsc.load_gather(big_v, [idx])

        pltpu.sync_copy(oute1_v,
                        outT_hbm.at[wid, pl.ds(c0, P1_CH)])

    g = wid % CG
    h = wid // CG
    rbase = h * RPG
    cbase = g * CPG

    tds = [pltpu.async_copy(
        textT_hbm.at[cbase + j],
        big_v.at[pl.ds(j * TVOCAB, TVOCAB)], sem)
        for j in range(CPG)]
    for d in tds:
        d.wait()

    lf = jnp.full((16,), float(L), dtype=jnp.float32)
    onef = jnp.full((16,), 1.0, dtype=jnp.float32)
    cvecs = [jnp.full((16,), j * TVOCAB, dtype=jnp.int32) for j in range(CPG)]
    t0s = [big_v[pl.ds(j * TVOCAB, 16)][0] for j in range(CPG)]

    for ch in range(N2CH):
        r0 = rbase + ch * P2_CH
        pltpu.sync_copy(idsT_hbm.at[:, pl.ds(r0, P2_CH)], ids_v)

        @plsc.parallel_loop(0, P2_CH // 16, unroll=1)
        def e2grp(gg):
            rr = gg * 16
            idvecs = [ids_v[t, pl.ds(rr, 16)] for t in range(L)]
            cnt = _tree_sum([jnp.where(v != 0, 1.0, 0.0) for v in idvecs])
            n0f = lf - cnt
            recip = 1.0 / jnp.maximum(cnt, onef)
            for j in range(CPG):
                acc = _tree_sum([
                    plsc.load_gather(big_v, [v + cvecs[j]]) for v in idvecs])
                acc_v[j, pl.ds(rr, 16)] = (acc - n0f * t0s[j]) * recip

        for j in range(CPG):
            pltpu.sync_copy(acc_v.at[j],
                            outT_hbm.at[EMB + cbase + j, pl.ds(r0, P2_CH)])


def _tree_sum(vs):
    while len(vs) > 1:
        nxt = [vs[i] + vs[i + 1] for i in range(0, len(vs) - 1, 2)]
        if len(vs) % 2:
            nxt.append(vs[-1])
        vs = nxt
    return vs[0]


def kernel(anime_ids, token_ids, name_table, text_table):
    aidx = anime_ids.astype(jnp.int32)
    idsT = token_ids.astype(jnp.int32).T
    nameT = jnp.pad(name_table.T, ((0, 0), (0, VPAD - VOCAB1)))
    textT = text_table.T

    mesh = plsc.VectorSubcoreMesh(core_axis_name="c", subcore_axis_name="s")
    run = functools.partial(
        pl.kernel, mesh=mesh,
        out_type=jax.ShapeDtypeStruct((2 * EMB, B), jnp.float32),
        compiler_params=pltpu.CompilerParams(
            needs_layout_passes=False, use_tc_tiling_on_sc=False),
        scratch_types=[
            pltpu.VMEM((VPAD,), jnp.float32),
            pltpu.VMEM((L, P2_CH), jnp.int32),
            pltpu.VMEM((P1_CH,), jnp.int32),
            pltpu.VMEM((P1_CH,), jnp.float32),
            pltpu.VMEM((CPG, P2_CH), jnp.float32),
            pltpu.SemaphoreType.DMA,
            pltpu.SemaphoreType.DMA,
        ],
    )(_body)
    outT = run(aidx, idsT, nameT, textT)
    return outT.T


# speedup vs baseline: 19.5986x; 1.1476x over previous
import functools

import jax
import jax.numpy as jnp
from jax import lax
from jax.experimental import pallas as pl
from jax.experimental.pallas import tpu as pltpu
from jax.experimental.pallas import tpu_sc as plsc

B = 16384
L = 20
LH = L // 2
EMB = 32
VOCAB1 = 100001
VPAD = 100008
NAMEF = EMB * VOCAB1
TVOCAB = 10000
NC = 2
NS = 16
NW = NC * NS

P1_CH = 2048
N1CH = B // P1_CH
CG = 4
CPG = EMB // CG
RG = NW // CG
RPG = B // RG
P2_CH = 256
N2CH = RPG // P2_CH


def _tree_sum(vs):
    while len(vs) > 1:
        nxt = [vs[i] + vs[i + 1] for i in range(0, len(vs) - 1, 2)]
        if len(vs) % 2:
            nxt.append(vs[-1])
        vs = nxt
    return vs[0]


def _body(aidx_hbm, idsT_hbm, nameF_hbm, textT_hbm, outT_hbm,
          big_v, ids_v, aidx_v, oute1_v, acc_v,
          isem0, isem1, osem0, osem1, bsem):
    wid = lax.axis_index("s") * NC + lax.axis_index("c")

    off = wid * VOCAB1
    sh = lax.rem(off, 8)
    astart = pl.multiple_of(off - sh, 8)
    big_dma = pltpu.async_copy(nameF_hbm.at[pl.ds(astart, VPAD)], big_v, bsem)
    shv = jnp.full((16,), 0, jnp.int32) + sh

    isems = [isem0, isem1]
    osems = [osem0, osem1]
    idmas = [pltpu.async_copy(
        aidx_hbm.at[pl.ds(p * P1_CH, P1_CH)], aidx_v.at[p], isems[p])
        for p in range(2)]
    big_dma.wait()
    odmas = [None, None]
    for ch in range(N1CH):
        p = ch % 2
        idmas[p].wait()
        if odmas[p] is not None:
            odmas[p].wait()

        @plsc.parallel_loop(0, P1_CH // 16, unroll=2)
        def e1grp(g):
            rr = g * 16
            idx = aidx_v[p, pl.ds(rr, 16)] + shv
            oute1_v[p, pl.ds(rr, 16)] = plsc.load_gather(big_v, [idx])

        odmas[p] = pltpu.async_copy(
            oute1_v.at[p], outT_hbm.at[wid, pl.ds(ch * P1_CH, P1_CH)],
            osems[p])
        if ch + 2 < N1CH:
            idmas[p] = pltpu.async_copy(
                aidx_hbm.at[pl.ds((ch + 2) * P1_CH, P1_CH)], aidx_v.at[p],
                isems[p])
    for d in odmas:
        d.wait()

    g = wid % CG
    h = wid // CG
    rbase = h * RPG
    cbase = g * CPG

    tds = [pltpu.async_copy(
        textT_hbm.at[cbase + j],
        big_v.at[pl.ds(j * TVOCAB, TVOCAB)], bsem)
        for j in range(CPG)]
    for d in tds:
        d.wait()

    lf = jnp.full((16,), float(L), dtype=jnp.float32)
    onef = jnp.full((16,), 1.0, dtype=jnp.float32)
    cvecs = [jnp.full((16,), j * TVOCAB, dtype=jnp.int32) for j in range(CPG)]
    t0s = [big_v[pl.ds(j * TVOCAB, 16)][0] for j in range(CPG)]

    idmas = [pltpu.async_copy(
        idsT_hbm.at[:, pl.ds(rbase + p * P2_CH, P2_CH)], ids_v.at[p],
        isems[p]) for p in range(2)]
    odmas = [None, None]
    for ch in range(N2CH):
        p = ch % 2
        r0 = rbase + ch * P2_CH
        idmas[p].wait()
        if odmas[p] is not None:
            for d in odmas[p]:
                d.wait()

        @plsc.parallel_loop(0, P2_CH // 16, unroll=1)
        def e2grp(gg):
            rr = gg * 16
            iv0 = [ids_v[p, t, pl.ds(rr, 16)] for t in range(LH)]
            iv1 = [ids_v[p, t, pl.ds(rr, 16)] for t in range(LH, L)]
            cnt = (_tree_sum([jnp.where(v != 0, 1.0, 0.0) for v in iv0])
                   + _tree_sum([jnp.where(v != 0, 1.0, 0.0) for v in iv1]))
            n0f = lf - cnt
            recip = 1.0 / jnp.maximum(cnt, onef)
            for j in range(CPG):
                acc = (_tree_sum([plsc.load_gather(big_v, [v + cvecs[j]])
                                  for v in iv0])
                       + _tree_sum([plsc.load_gather(big_v, [v + cvecs[j]])
                                    for v in iv1]))
                acc_v[p, j, pl.ds(rr, 16)] = (acc - n0f * t0s[j]) * recip

        odmas[p] = [pltpu.async_copy(
            acc_v.at[p, j],
            outT_hbm.at[EMB + cbase + j, pl.ds(r0, P2_CH)], osems[p])
            for j in range(CPG)]
        if ch + 2 < N2CH:
            idmas[p] = pltpu.async_copy(
                idsT_hbm.at[:, pl.ds(rbase + (ch + 2) * P2_CH, P2_CH)],
                ids_v.at[p], isems[p])
    for ds in odmas:
        for d in ds:
            d.wait()


def kernel(anime_ids, token_ids, name_table, text_table):
    aidx = anime_ids.astype(jnp.int32)
    idsT = token_ids.astype(jnp.int32).T
    nameF = name_table.T.reshape(NAMEF)
    textT = text_table.T

    mesh = plsc.VectorSubcoreMesh(core_axis_name="c", subcore_axis_name="s")
    run = functools.partial(
        pl.kernel, mesh=mesh,
        out_type=jax.ShapeDtypeStruct((2 * EMB, B), jnp.float32),
        compiler_params=pltpu.CompilerParams(
            needs_layout_passes=False, use_tc_tiling_on_sc=False),
        scratch_types=[
            pltpu.VMEM((VPAD,), jnp.float32),
            pltpu.VMEM((2, L, P2_CH), jnp.int32),
            pltpu.VMEM((2, P1_CH), jnp.int32),
            pltpu.VMEM((2, P1_CH), jnp.float32),
            pltpu.VMEM((2, CPG, P2_CH), jnp.float32),
            pltpu.SemaphoreType.DMA,
            pltpu.SemaphoreType.DMA,
            pltpu.SemaphoreType.DMA,
            pltpu.SemaphoreType.DMA,
            pltpu.SemaphoreType.DMA,
        ],
    )(_body)
    outT = run(aidx, idsT, nameF, textT)
    return outT.T
